# baseline (speedup 1.0000x reference)
import functools

import jax
import jax.numpy as jnp
from jax import lax
from jax.experimental import pallas as pl
from jax.experimental.pallas import tpu as pltpu
from jax.experimental.pallas import tpu_sc as plsc

KNN = 20
N = 1024
BATCH = 32
NEG = float("-inf")
_CNT = BATCH * N * KNN
_HIGH = jax.lax.Precision.HIGHEST


def _mm(a, b):
    return jax.lax.dot_general(a, b, (((1,), (0,)), ((), ())),
                               preferred_element_type=jnp.float32,
                               precision=_HIGH)


def _score(h):
    g = jax.lax.dot_general(h, h, (((0,), (0,)), ((), ())),
                            preferred_element_type=jnp.float32,
                            precision=_HIGH)
    hh = h * h
    ones = jnp.ones((h.shape[0], 8), jnp.float32)
    xxc = jax.lax.dot_general(hh, ones, (((0,), (0,)), ((), ())),
                              preferred_element_type=jnp.float32,
                              precision=_HIGH)
    return 2.0 * g - xxc[:, 0:1]


def _gather_rows(tbl, idxrow):
    t_rows = tbl.shape[0]
    idx = jnp.broadcast_to(idxrow, (t_rows, N))
    loc = idx & 127
    hi = idx >> 7
    acc = jnp.zeros((t_rows, N), jnp.float32)
    for cc in range(N // 128):
        src = jax.lax.slice(tbl, (0, cc * 128), (t_rows, cc * 128 + 128))
        g = jnp.take_along_axis(src, loc, axis=1)
        acc = jnp.where(hi == cc, g, acc)
    return acc


def _knn_round(d_ref, sub, m):
    dcur = d_ref[...]
    imin = jnp.min(jnp.where(dcur == m, sub, N), axis=0, keepdims=True)
    m_next = jnp.max(jnp.where(dcur < m, dcur, NEG), axis=0, keepdims=True)
    return imin, m_next


def _bn_coeffs(asum_ref, asq_ref, g_ref, b_ref):
    mean = jnp.sum(asum_ref[...], axis=1, keepdims=True) * (1.0 / _CNT)
    ex2 = jnp.sum(asq_ref[...], axis=1, keepdims=True) * (1.0 / _CNT)
    var = ex2 - mean * mean
    inv = jax.lax.rsqrt(var + 1e-5)
    scale = g_ref[...] * inv
    shift = b_ref[...] - mean * scale
    return scale, shift


def _acc_stats(b, asum_ref, asq_ref, ssum, ssq):
    c = ssum.shape[0]
    ps = ssum.reshape(c, 8, 128).sum(axis=1)
    pq = ssq.reshape(c, 8, 128).sum(axis=1)

    @pl.when(b == 0)
    def _():
        asum_ref[...] = jnp.zeros_like(asum_ref)
        asq_ref[...] = jnp.zeros_like(asq_ref)

    asum_ref[...] += ps
    asq_ref[...] += pq


def _f0_kernel(x_ref, n_ref, a_ref, bm_ref, c_ref, dm_ref, wd_ref, wo_ref,
               out_ref, asum_ref, asq_ref, d_ref):
    b = pl.program_id(0)
    xyz = x_ref[0]
    nrm = n_ref[0]
    d_ref[...] = _score(xyz)
    ps = _mm(a_ref[...], xyz) + _mm(dm_ref[...], nrm)
    q = _mm(bm_ref[...], xyz) + _mm(c_ref[...], nrm)
    wd = wd_ref[...]
    wo = wo_ref[...]
    tbl = jnp.concatenate([ps, xyz, nrm], axis=0)
    sub = jax.lax.broadcasted_iota(jnp.int32, (N, N), 0)
    m0 = jnp.max(d_ref[...], axis=0, keepdims=True)

    def body(t, carry):
        ssum, ssq, m = carry
        imin, m_next = _knn_round(d_ref, sub, m)
        g = _gather_rows(tbl, imin)
        gj = g[0:16]
        xyzj = g[16:19]
        nj = g[19:22]
        diff = xyzj - xyz
        dist = jnp.sqrt(jnp.sum(diff * diff, axis=0, keepdims=True) + 1e-12)
        dot = jnp.sum(nrm * nj, axis=0, keepdims=True)
        out_t = gj + q + wd * dist + wo * dot
        out_ref[0, t] = out_t
        return ssum + out_t, ssq + out_t * out_t, m_next

    z = jnp.zeros((16, N), jnp.float32)
    ssum, ssq, _ = jax.lax.fori_loop(0, KNN, body, (z, z, m0))
    _acc_stats(b, asum_ref, asq_ref, ssum, ssq)


def _mid_body(has_res, write_res, cout,
              prev_ref, rprev_ref, asum_p_ref, asq_p_ref, g_ref, b_ref,
              wa_ref, wdlt_ref, rout_ref, out_ref, asum_ref, asq_ref,
              d_ref):
    b = pl.program_id(0)
    scale, shift = _bn_coeffs(asum_p_ref, asq_p_ref, g_ref, b_ref)

    def pool_body(t, hmax):
        y = prev_ref[0, t] * scale + shift
        act = jnp.where(y >= 0, y, 0.2 * y)
        if has_res:
            act = act + rprev_ref[0, t]
        if write_res:
            rout_ref[0, t] = act
        return jnp.maximum(hmax, act)

    h = jax.lax.fori_loop(0, KNN, pool_body, jnp.full((16, N), NEG, jnp.float32))

    d_ref[...] = _score(h)
    u = _mm(wa_ref[...], h)
    v = _mm(wdlt_ref[...], h)
    sub = jax.lax.broadcasted_iota(jnp.int32, (N, N), 0)
    m0 = jnp.max(d_ref[...], axis=0, keepdims=True)

    if cout == 16:
        def body(t, carry):
            ssum, ssq, m = carry
            imin, m_next = _knn_round(d_ref, sub, m)
            out_t = _gather_rows(u, imin) + v
            out_ref[0, t] = out_t
            return ssum + out_t, ssq + out_t * out_t, m_next

        z = jnp.zeros((16, N), jnp.float32)
        ssum, ssq, _ = jax.lax.fori_loop(0, KNN, body, (z, z, m0))
    else:
        def body(t, carry):
            ssum, ssq, pmax, m = carry
            imin, m_next = _knn_round(d_ref, sub, m)
            out_t = _gather_rows(u, imin) + v
            return (ssum + out_t, ssq + out_t * out_t,
                    jnp.maximum(pmax, out_t), m_next)

        z = jnp.zeros((cout, N), jnp.float32)
        ssum, ssq, pmax, _ = jax.lax.fori_loop(
            0, KNN, body, (z, z, jnp.full((cout, N), NEG, jnp.float32), m0))
        out_ref[0] = pmax
    _acc_stats(b, asum_ref, asq_ref, ssum, ssq)


def _mid_kernel_first(prev_ref, asum_p_ref, asq_p_ref, g_ref, b_ref,
                      wa_ref, wdlt_ref, rout_ref, out_ref,
                      asum_ref, asq_ref, d_ref):
    _mid_body(False, True, 16, prev_ref, None, asum_p_ref, asq_p_ref,
              g_ref, b_ref, wa_ref, wdlt_ref, rout_ref, out_ref,
              asum_ref, asq_ref, d_ref)


def _mid_kernel_res(prev_ref, rprev_ref, asum_p_ref, asq_p_ref,
                    g_ref, b_ref, wa_ref, wdlt_ref, rout_ref, out_ref,
                    asum_ref, asq_ref, d_ref):
    _mid_body(True, True, 16, prev_ref, rprev_ref, asum_p_ref, asq_p_ref,
              g_ref, b_ref, wa_ref, wdlt_ref, rout_ref, out_ref,
              asum_ref, asq_ref, d_ref)


def _mid_kernel_last(prev_ref, rprev_ref, asum_p_ref, asq_p_ref,
                     g_ref, b_ref, wa_ref, wdlt_ref, out_ref,
                     asum_ref, asq_ref, d_ref):
    _mid_body(True, False, 128, prev_ref, rprev_ref, asum_p_ref, asq_p_ref,
              g_ref, b_ref, wa_ref, wdlt_ref, None, out_ref,
              asum_ref, asq_ref, d_ref)


_NW = 32
_EDGES = BATCH * KNN * N
_EPW = _EDGES // _NW
_CHUNK = 5120


def _f1a_kernel(prev_ref, asum_p_ref, asq_p_ref, g_ref, b_ref, wa_ref,
                wdlt_ref, rout_ref, idx_ref, ut_ref, v_ref, d_ref):
    b = pl.program_id(0)
    scale, shift = _bn_coeffs(asum_p_ref, asq_p_ref, g_ref, b_ref)

    def pool_body(t, hmax):
        y = prev_ref[0, t] * scale + shift
        act = jnp.where(y >= 0, y, 0.2 * y)
        rout_ref[0, t] = act
        return jnp.maximum(hmax, act)

    h = jax.lax.fori_loop(0, KNN, pool_body,
                          jnp.full((16, N), NEG, jnp.float32))
    d_ref[...] = _score(h)
    u = _mm(wa_ref[...], h)
    v_ref[0] = _mm(wdlt_ref[...], h)
    ut_ref[0] = jnp.transpose(u)
    sub = jax.lax.broadcasted_iota(jnp.int32, (N, N), 0)
    m0 = jnp.max(d_ref[...], axis=0, keepdims=True)
    base = b * N

    def body(t, m):
        imin, m_next = _knn_round(d_ref, sub, m)
        idx_ref[0, t] = imin + base
        return m_next

    jax.lax.fori_loop(0, KNN, body, m0)


def _sc_gather_body(table_ref, idx_ref, out_ref, idx_v, rows_v, sem):
    wid = lax.axis_index("s") * 2 + lax.axis_index("c")
    base = wid * _EPW
    for i in range(_EPW // _CHUNK):
        off = base + i * _CHUNK
        pltpu.sync_copy(idx_ref.at[pl.ds(off, _CHUNK)], idx_v)
        pltpu.async_copy(table_ref.at[idx_v], rows_v, sem).wait()
        pltpu.sync_copy(rows_v, out_ref.at[pl.ds(off, _CHUNK)])


def _f1b_kernel(g_ref, v_ref, out_ref, asum_ref, asq_ref):
    b = pl.program_id(0)
    v = v_ref[0]

    def body(t, carry):
        ssum, ssq = carry
        out_t = jnp.transpose(g_ref[0, t]) + v
        out_ref[0, t] = out_t
        return ssum + out_t, ssq + out_t * out_t

    z = jnp.zeros((16, N), jnp.float32)
    ssum, ssq = jax.lax.fori_loop(0, KNN, body, (z, z))
    _acc_stats(b, asum_ref, asq_ref, ssum, ssq)


def _f4_kernel(pm_ref, asum_ref, asq_ref, g_ref, b_ref, pooled_ref):
    scale, shift = _bn_coeffs(asum_ref, asq_ref, g_ref, b_ref)
    y = pm_ref[0] * scale + shift
    act = jnp.where(y >= 0, y, 0.2 * y)
    pooled_ref[0] = jnp.mean(act, axis=1, keepdims=True)


def _f5_kernel(pooled_ref, wl_ref, bl_ref, out_ref):
    logits = jax.lax.dot_general(pooled_ref[...], wl_ref[...],
                                 (((1,), (1,)), ((), ())),
                                 preferred_element_type=jnp.float32,
                                 precision=_HIGH)
    logits = logits + bl_ref[0:1, :]
    m = jnp.max(logits, axis=1, keepdims=True)
    zz = logits - m
    lse = jnp.log(jnp.sum(jnp.exp(zz), axis=1, keepdims=True))
    out_ref[...] = zz - lse


def _full(shape):
    nd = len(shape)
    return pl.BlockSpec(shape, lambda b, _n=nd: (0,) * _n)


def _per_b(shape):
    nd = len(shape)
    return pl.BlockSpec((1,) + shape, lambda b, _n=nd: (b,) + (0,) * _n)


_EDGE = jax.ShapeDtypeStruct((BATCH, KNN, 16, N), jnp.float32)
_STAT16 = jax.ShapeDtypeStruct((16, 128), jnp.float32)
_STAT128 = jax.ShapeDtypeStruct((128, 128), jnp.float32)


def kernel(x, n, W1, g1, b1, W2, g2, b2, W3, g3, b3, W4, g4, b4, Wl, bl):
    f32 = jnp.float32
    a1 = W1[:, 0:3]
    bm1 = W1[:, 3:6] - a1
    c1 = W1[:, 6:9]
    dm1 = W1[:, 9:12]
    wd1 = W1[:, 12:13]
    wo1 = W1[:, 13:14]
    wa2, wdl2 = W2[:, 0:16], W2[:, 16:32] - W2[:, 0:16]
    wa3, wdl3 = W3[:, 0:16], W3[:, 16:32] - W3[:, 0:16]
    wa4, wdl4 = W4[:, 0:16], W4[:, 16:32] - W4[:, 0:16]
    g1c, b1c = g1.reshape(16, 1), b1.reshape(16, 1)
    g2c, b2c = g2.reshape(16, 1), b2.reshape(16, 1)
    g3c, b3c = g3.reshape(16, 1), b3.reshape(16, 1)
    g4c, b4c = g4.reshape(128, 1), b4.reshape(128, 1)
    bl8 = jnp.broadcast_to(bl.reshape(1, 40), (8, 40))

    scratch = [pltpu.VMEM((N, N), f32)]

    out1, s1, q1 = pl.pallas_call(
        _f0_kernel,
        grid=(BATCH,),
        in_specs=[_per_b((3, N)), _per_b((3, N)), _full((16, 3)),
                  _full((16, 3)), _full((16, 3)), _full((16, 3)),
                  _full((16, 1)), _full((16, 1))],
        out_specs=[_per_b((KNN, 16, N)), _full((16, 128)), _full((16, 128))],
        out_shape=[_EDGE, _STAT16, _STAT16],
        scratch_shapes=scratch,
    )(x, n, a1, bm1, c1, dm1, wd1, wo1)

    r1, idxg, ut, v2 = pl.pallas_call(
        _f1a_kernel,
        grid=(BATCH,),
        in_specs=[_per_b((KNN, 16, N)),
                  _full((16, 128)), _full((16, 128)),
                  _full((16, 1)), _full((16, 1)),
                  _full((16, 16)), _full((16, 16))],
        out_specs=[_per_b((KNN, 16, N)), _per_b((KNN, 1, N)),
                   _per_b((N, 16)), _per_b((16, N))],
        out_shape=[_EDGE,
                   jax.ShapeDtypeStruct((BATCH, KNN, 1, N), jnp.int32),
                   jax.ShapeDtypeStruct((BATCH, N, 16), jnp.float32),
                   jax.ShapeDtypeStruct((BATCH, 16, N), jnp.float32)],
        scratch_shapes=scratch,
    )(out1, s1, q1, g1c, b1c, wa2, wdl2)

    mesh = plsc.VectorSubcoreMesh(core_axis_name="c", subcore_axis_name="s")
    scg = pl.kernel(
        _sc_gather_body,
        mesh=mesh,
        out_type=jax.ShapeDtypeStruct((_EDGES, 16), jnp.float32),
        scratch_types=[pltpu.VMEM((_CHUNK,), jnp.int32),
                       pltpu.VMEM((_CHUNK, 16), jnp.float32),
                       pltpu.SemaphoreType.DMA],
        compiler_params=pltpu.CompilerParams(use_tc_tiling_on_sc=False),
    )
    gath = scg(ut.reshape(BATCH * N, 16), idxg.reshape(_EDGES))

    out2, s2, q2 = pl.pallas_call(
        _f1b_kernel,
        grid=(BATCH,),
        in_specs=[_per_b((KNN, N, 16)), _per_b((16, N))],
        out_specs=[_per_b((KNN, 16, N)), _full((16, 128)), _full((16, 128))],
        out_shape=[_EDGE, _STAT16, _STAT16],
    )(gath.reshape(BATCH, KNN, N, 16), v2)

    r2, out3, s3, q3 = pl.pallas_call(
        _mid_kernel_res,
        grid=(BATCH,),
        in_specs=[_per_b((KNN, 16, N)), _per_b((KNN, 16, N)),
                  _full((16, 128)), _full((16, 128)),
                  _full((16, 1)), _full((16, 1)),
                  _full((16, 16)), _full((16, 16))],
        out_specs=[_per_b((KNN, 16, N)), _per_b((KNN, 16, N)),
                   _full((16, 128)), _full((16, 128))],
        out_shape=[_EDGE, _EDGE, _STAT16, _STAT16],
        scratch_shapes=scratch,
    )(out2, r1, s2, q2, g2c, b2c, wa3, wdl3)

    pm4, s4, q4 = pl.pallas_call(
        _mid_kernel_last,
        grid=(BATCH,),
        in_specs=[_per_b((KNN, 16, N)), _per_b((KNN, 16, N)),
                  _full((16, 128)), _full((16, 128)),
                  _full((16, 1)), _full((16, 1)),
                  _full((128, 16)), _full((128, 16))],
        out_specs=[_per_b((128, N)), _full((128, 128)), _full((128, 128))],
        out_shape=[jax.ShapeDtypeStruct((BATCH, 128, N), f32),
                   _STAT128, _STAT128],
        scratch_shapes=scratch,
    )(out3, r2, s3, q3, g3c, b3c, wa4, wdl4)

    pooled = pl.pallas_call(
        _f4_kernel,
        grid=(BATCH,),
        in_specs=[_per_b((128, N)), _full((128, 128)), _full((128, 128)),
                  _full((128, 1)), _full((128, 1))],
        out_specs=_per_b((128, 1)),
        out_shape=jax.ShapeDtypeStruct((BATCH, 128, 1), f32),
    )(pm4, s4, q4, g4c, b4c)

    pooled2 = pooled.reshape(BATCH, 128)
    out = pl.pallas_call(
        _f5_kernel,
        in_specs=[pl.BlockSpec((BATCH, 128), lambda: (0, 0)),
                  pl.BlockSpec((40, 128), lambda: (0, 0)),
                  pl.BlockSpec((8, 40), lambda: (0, 0))],
        out_specs=pl.BlockSpec((BATCH, 40), lambda: (0, 0)),
        out_shape=jax.ShapeDtypeStruct((BATCH, 40), f32),
    )(pooled2, Wl, bl8)
    return out


# speedup vs baseline: 1.0883x; 1.0883x over previous
import functools

import jax
import jax.numpy as jnp
from jax import lax
from jax.experimental import pallas as pl
from jax.experimental.pallas import tpu as pltpu
from jax.experimental.pallas import tpu_sc as plsc

KNN = 20
N = 1024
BATCH = 32
NEG = float("-inf")
_CNT = BATCH * N * KNN
_HIGH = jax.lax.Precision.HIGHEST


def _mm(a, b):
    return jax.lax.dot_general(a, b, (((1,), (0,)), ((), ())),
                               preferred_element_type=jnp.float32,
                               precision=_HIGH)


def _score(h):
    g = jax.lax.dot_general(h, h, (((0,), (0,)), ((), ())),
                            preferred_element_type=jnp.float32,
                            precision=_HIGH)
    hh = h * h
    ones = jnp.ones((h.shape[0], 8), jnp.float32)
    xxc = jax.lax.dot_general(hh, ones, (((0,), (0,)), ((), ())),
                              preferred_element_type=jnp.float32,
                              precision=_HIGH)
    return 2.0 * g - xxc[:, 0:1]


def _gather_rows(tbl, idxrow):
    t_rows = tbl.shape[0]
    idx = jnp.broadcast_to(idxrow, (t_rows, N))
    loc = idx & 127
    hi = idx >> 7
    acc = jnp.zeros((t_rows, N), jnp.float32)
    for cc in range(N // 128):
        src = jax.lax.slice(tbl, (0, cc * 128), (t_rows, cc * 128 + 128))
        g = jnp.take_along_axis(src, loc, axis=1)
        acc = jnp.where(hi == cc, g, acc)
    return acc


def _knn_round(d_ref, sub, m):
    dcur = d_ref[...]
    imin = jnp.min(jnp.where(dcur == m, sub, N), axis=0, keepdims=True)
    m_next = jnp.max(jnp.where(dcur < m, dcur, NEG), axis=0, keepdims=True)
    return imin, m_next


def _bn_coeffs(asum_ref, asq_ref, g_ref, b_ref):
    mean = jnp.sum(asum_ref[...], axis=1, keepdims=True) * (1.0 / _CNT)
    ex2 = jnp.sum(asq_ref[...], axis=1, keepdims=True) * (1.0 / _CNT)
    var = ex2 - mean * mean
    inv = jax.lax.rsqrt(var + 1e-5)
    scale = g_ref[...] * inv
    shift = b_ref[...] - mean * scale
    return scale, shift


def _acc_stats(b, asum_ref, asq_ref, ssum, ssq):
    c = ssum.shape[0]
    ps = ssum.reshape(c, 8, 128).sum(axis=1)
    pq = ssq.reshape(c, 8, 128).sum(axis=1)

    @pl.when(b == 0)
    def _():
        asum_ref[...] = jnp.zeros_like(asum_ref)
        asq_ref[...] = jnp.zeros_like(asq_ref)

    asum_ref[...] += ps
    asq_ref[...] += pq


def _f0_kernel(x_ref, n_ref, a_ref, bm_ref, c_ref, dm_ref, wd_ref, wo_ref,
               out_ref, asum_ref, asq_ref, d_ref):
    b = pl.program_id(0)
    xyz = x_ref[0]
    nrm = n_ref[0]
    d_ref[...] = _score(xyz)
    ps = _mm(a_ref[...], xyz) + _mm(dm_ref[...], nrm)
    q = _mm(bm_ref[...], xyz) + _mm(c_ref[...], nrm)
    wd = wd_ref[...]
    wo = wo_ref[...]
    tbl = jnp.concatenate([ps, xyz, nrm], axis=0)
    sub = jax.lax.broadcasted_iota(jnp.int32, (N, N), 0)
    m0 = jnp.max(d_ref[...], axis=0, keepdims=True)

    def body(t, carry):
        ssum, ssq, m = carry
        imin, m_next = _knn_round(d_ref, sub, m)
        g = _gather_rows(tbl, imin)
        gj = g[0:16]
        xyzj = g[16:19]
        nj = g[19:22]
        diff = xyzj - xyz
        dist = jnp.sqrt(jnp.sum(diff * diff, axis=0, keepdims=True) + 1e-12)
        dot = jnp.sum(nrm * nj, axis=0, keepdims=True)
        out_t = gj + q + wd * dist + wo * dot
        out_ref[0, t] = out_t
        return ssum + out_t, ssq + out_t * out_t, m_next

    z = jnp.zeros((16, N), jnp.float32)
    ssum, ssq, _ = jax.lax.fori_loop(0, KNN, body, (z, z, m0), unroll=2)
    _acc_stats(b, asum_ref, asq_ref, ssum, ssq)


def _mid_body(has_res, write_res, cout,
              prev_ref, rprev_ref, asum_p_ref, asq_p_ref, g_ref, b_ref,
              wa_ref, wdlt_ref, rout_ref, out_ref, asum_ref, asq_ref,
              d_ref):
    b = pl.program_id(0)
    scale, shift = _bn_coeffs(asum_p_ref, asq_p_ref, g_ref, b_ref)

    def pool_body(t, hmax):
        y = prev_ref[0, t] * scale + shift
        act = jnp.where(y >= 0, y, 0.2 * y)
        if has_res:
            act = act + rprev_ref[0, t]
        if write_res:
            rout_ref[0, t] = act
        return jnp.maximum(hmax, act)

    h = jax.lax.fori_loop(0, KNN, pool_body, jnp.full((16, N), NEG, jnp.float32))

    d_ref[...] = _score(h)
    u = _mm(wa_ref[...], h)
    v = _mm(wdlt_ref[...], h)
    sub = jax.lax.broadcasted_iota(jnp.int32, (N, N), 0)
    m0 = jnp.max(d_ref[...], axis=0, keepdims=True)

    if cout == 16:
        def body(t, carry):
            ssum, ssq, m = carry
            imin, m_next = _knn_round(d_ref, sub, m)
            out_t = _gather_rows(u, imin) + v
            out_ref[0, t] = out_t
            return ssum + out_t, ssq + out_t * out_t, m_next

        z = jnp.zeros((16, N), jnp.float32)
        ssum, ssq, _ = jax.lax.fori_loop(0, KNN, body, (z, z, m0), unroll=2)
    else:
        def body(t, carry):
            ssum, ssq, pmax, m = carry
            imin, m_next = _knn_round(d_ref, sub, m)
            out_t = _gather_rows(u, imin) + v
            return (ssum + out_t, ssq + out_t * out_t,
                    jnp.maximum(pmax, out_t), m_next)

        z = jnp.zeros((cout, N), jnp.float32)
        ssum, ssq, pmax, _ = jax.lax.fori_loop(
            0, KNN, body, (z, z, jnp.full((cout, N), NEG, jnp.float32), m0),
            unroll=2)
        out_ref[0] = pmax
    _acc_stats(b, asum_ref, asq_ref, ssum, ssq)


def _mid_kernel_first(prev_ref, asum_p_ref, asq_p_ref, g_ref, b_ref,
                      wa_ref, wdlt_ref, rout_ref, out_ref,
                      asum_ref, asq_ref, d_ref):
    _mid_body(False, True, 16, prev_ref, None, asum_p_ref, asq_p_ref,
              g_ref, b_ref, wa_ref, wdlt_ref, rout_ref, out_ref,
              asum_ref, asq_ref, d_ref)


def _mid_kernel_res(prev_ref, rprev_ref, asum_p_ref, asq_p_ref,
                    g_ref, b_ref, wa_ref, wdlt_ref, rout_ref, out_ref,
                    asum_ref, asq_ref, d_ref):
    _mid_body(True, True, 16, prev_ref, rprev_ref, asum_p_ref, asq_p_ref,
              g_ref, b_ref, wa_ref, wdlt_ref, rout_ref, out_ref,
              asum_ref, asq_ref, d_ref)


def _mid_kernel_last(prev_ref, rprev_ref, asum_p_ref, asq_p_ref,
                     g_ref, b_ref, wa_ref, wdlt_ref, out_ref,
                     asum_ref, asq_ref, d_ref):
    _mid_body(True, False, 128, prev_ref, rprev_ref, asum_p_ref, asq_p_ref,
              g_ref, b_ref, wa_ref, wdlt_ref, None, out_ref,
              asum_ref, asq_ref, d_ref)


_NW = 32
_EDGES = BATCH * KNN * N
_EPW = _EDGES // _NW
_CHUNK = 5120


def _f1a_kernel(prev_ref, asum_p_ref, asq_p_ref, g_ref, b_ref, wa_ref,
                wdlt_ref, rout_ref, idx_ref, ut_ref, v_ref, d_ref):
    b = pl.program_id(0)
    scale, shift = _bn_coeffs(asum_p_ref, asq_p_ref, g_ref, b_ref)

    def pool_body(t, hmax):
        y = prev_ref[0, t] * scale + shift
        act = jnp.where(y >= 0, y, 0.2 * y)
        rout_ref[0, t] = act
        return jnp.maximum(hmax, act)

    h = jax.lax.fori_loop(0, KNN, pool_body,
                          jnp.full((16, N), NEG, jnp.float32))
    d_ref[...] = _score(h)
    u = _mm(wa_ref[...], h)
    v_ref[0] = _mm(wdlt_ref[...], h)
    ut_ref[0] = jnp.transpose(u)
    sub = jax.lax.broadcasted_iota(jnp.int32, (N, N), 0)
    m0 = jnp.max(d_ref[...], axis=0, keepdims=True)
    base = b * N

    def body(t, m):
        imin, m_next = _knn_round(d_ref, sub, m)
        idx_ref[0, t] = imin + base
        return m_next

    jax.lax.fori_loop(0, KNN, body, m0, unroll=2)


def _sc_gather_body(table_ref, idx_ref, out_ref, idx_v, rows_v, sem):
    wid = lax.axis_index("s") * 2 + lax.axis_index("c")
    base = wid * _EPW
    for i in range(_EPW // _CHUNK):
        off = base + i * _CHUNK
        pltpu.sync_copy(idx_ref.at[pl.ds(off, _CHUNK)], idx_v)
        pltpu.async_copy(table_ref.at[idx_v], rows_v, sem).wait()
        pltpu.sync_copy(rows_v, out_ref.at[pl.ds(off, _CHUNK)])


def _f1b_kernel(g_ref, v_ref, out_ref, asum_ref, asq_ref):
    b = pl.program_id(0)
    v = v_ref[0]

    def body(t, carry):
        ssum, ssq = carry
        out_t = jnp.transpose(g_ref[0, t]) + v
        out_ref[0, t] = out_t
        return ssum + out_t, ssq + out_t * out_t

    z = jnp.zeros((16, N), jnp.float32)
    ssum, ssq = jax.lax.fori_loop(0, KNN, body, (z, z))
    _acc_stats(b, asum_ref, asq_ref, ssum, ssq)


def _f4_kernel(pm_ref, asum_ref, asq_ref, g_ref, b_ref, pooled_ref):
    scale, shift = _bn_coeffs(asum_ref, asq_ref, g_ref, b_ref)
    y = pm_ref[0] * scale + shift
    act = jnp.where(y >= 0, y, 0.2 * y)
    pooled_ref[0] = jnp.mean(act, axis=1, keepdims=True)


def _f5_kernel(pooled_ref, wl_ref, bl_ref, out_ref):
    logits = jax.lax.dot_general(pooled_ref[...], wl_ref[...],
                                 (((1,), (1,)), ((), ())),
                                 preferred_element_type=jnp.float32,
                                 precision=_HIGH)
    logits = logits + bl_ref[0:1, :]
    m = jnp.max(logits, axis=1, keepdims=True)
    zz = logits - m
    lse = jnp.log(jnp.sum(jnp.exp(zz), axis=1, keepdims=True))
    out_ref[...] = zz - lse


def _full(shape):
    nd = len(shape)
    return pl.BlockSpec(shape, lambda b, _n=nd: (0,) * _n)


def _per_b(shape):
    nd = len(shape)
    return pl.BlockSpec((1,) + shape, lambda b, _n=nd: (b,) + (0,) * _n)


_EDGE = jax.ShapeDtypeStruct((BATCH, KNN, 16, N), jnp.float32)
_STAT16 = jax.ShapeDtypeStruct((16, 128), jnp.float32)
_STAT128 = jax.ShapeDtypeStruct((128, 128), jnp.float32)


def kernel(x, n, W1, g1, b1, W2, g2, b2, W3, g3, b3, W4, g4, b4, Wl, bl):
    f32 = jnp.float32
    a1 = W1[:, 0:3]
    bm1 = W1[:, 3:6] - a1
    c1 = W1[:, 6:9]
    dm1 = W1[:, 9:12]
    wd1 = W1[:, 12:13]
    wo1 = W1[:, 13:14]
    wa2, wdl2 = W2[:, 0:16], W2[:, 16:32] - W2[:, 0:16]
    wa3, wdl3 = W3[:, 0:16], W3[:, 16:32] - W3[:, 0:16]
    wa4, wdl4 = W4[:, 0:16], W4[:, 16:32] - W4[:, 0:16]
    g1c, b1c = g1.reshape(16, 1), b1.reshape(16, 1)
    g2c, b2c = g2.reshape(16, 1), b2.reshape(16, 1)
    g3c, b3c = g3.reshape(16, 1), b3.reshape(16, 1)
    g4c, b4c = g4.reshape(128, 1), b4.reshape(128, 1)
    bl8 = jnp.broadcast_to(bl.reshape(1, 40), (8, 40))

    scratch = [pltpu.VMEM((N, N), f32)]

    out1, s1, q1 = pl.pallas_call(
        _f0_kernel,
        grid=(BATCH,),
        in_specs=[_per_b((3, N)), _per_b((3, N)), _full((16, 3)),
                  _full((16, 3)), _full((16, 3)), _full((16, 3)),
                  _full((16, 1)), _full((16, 1))],
        out_specs=[_per_b((KNN, 16, N)), _full((16, 128)), _full((16, 128))],
        out_shape=[_EDGE, _STAT16, _STAT16],
        scratch_shapes=scratch,
    )(x, n, a1, bm1, c1, dm1, wd1, wo1)

    r1, idxg, ut, v2 = pl.pallas_call(
        _f1a_kernel,
        grid=(BATCH,),
        in_specs=[_per_b((KNN, 16, N)),
                  _full((16, 128)), _full((16, 128)),
                  _full((16, 1)), _full((16, 1)),
                  _full((16, 16)), _full((16, 16))],
        out_specs=[_per_b((KNN, 16, N)), _per_b((KNN, 1, N)),
                   _per_b((N, 16)), _per_b((16, N))],
        out_shape=[_EDGE,
                   jax.ShapeDtypeStruct((BATCH, KNN, 1, N), jnp.int32),
                   jax.ShapeDtypeStruct((BATCH, N, 16), jnp.float32),
                   jax.ShapeDtypeStruct((BATCH, 16, N), jnp.float32)],
        scratch_shapes=scratch,
    )(out1, s1, q1, g1c, b1c, wa2, wdl2)

    mesh = plsc.VectorSubcoreMesh(core_axis_name="c", subcore_axis_name="s")
    scg = pl.kernel(
        _sc_gather_body,
        mesh=mesh,
        out_type=jax.ShapeDtypeStruct((_EDGES, 16), jnp.float32),
        scratch_types=[pltpu.VMEM((_CHUNK,), jnp.int32),
                       pltpu.VMEM((_CHUNK, 16), jnp.float32),
                       pltpu.SemaphoreType.DMA],
        compiler_params=pltpu.CompilerParams(use_tc_tiling_on_sc=False),
    )
    gath = scg(ut.reshape(BATCH * N, 16), idxg.reshape(_EDGES))

    out2, s2, q2 = pl.pallas_call(
        _f1b_kernel,
        grid=(BATCH,),
        in_specs=[_per_b((KNN, N, 16)), _per_b((16, N))],
        out_specs=[_per_b((KNN, 16, N)), _full((16, 128)), _full((16, 128))],
        out_shape=[_EDGE, _STAT16, _STAT16],
    )(gath.reshape(BATCH, KNN, N, 16), v2)

    r2, out3, s3, q3 = pl.pallas_call(
        _mid_kernel_res,
        grid=(BATCH,),
        in_specs=[_per_b((KNN, 16, N)), _per_b((KNN, 16, N)),
                  _full((16, 128)), _full((16, 128)),
                  _full((16, 1)), _full((16, 1)),
                  _full((16, 16)), _full((16, 16))],
        out_specs=[_per_b((KNN, 16, N)), _per_b((KNN, 16, N)),
                   _full((16, 128)), _full((16, 128))],
        out_shape=[_EDGE, _EDGE, _STAT16, _STAT16],
        scratch_shapes=scratch,
    )(out2, r1, s2, q2, g2c, b2c, wa3, wdl3)

    pm4, s4, q4 = pl.pallas_call(
        _mid_kernel_last,
        grid=(BATCH,),
        in_specs=[_per_b((KNN, 16, N)), _per_b((KNN, 16, N)),
                  _full((16, 128)), _full((16, 128)),
                  _full((16, 1)), _full((16, 1)),
                  _full((128, 16)), _full((128, 16))],
        out_specs=[_per_b((128, N)), _full((128, 128)), _full((128, 128))],
        out_shape=[jax.ShapeDtypeStruct((BATCH, 128, N), f32),
                   _STAT128, _STAT128],
        scratch_shapes=scratch,
    )(out3, r2, s3, q3, g3c, b3c, wa4, wdl4)

    pooled = pl.pallas_call(
        _f4_kernel,
        grid=(BATCH,),
        in_specs=[_per_b((128, N)), _full((128, 128)), _full((128, 128)),
                  _full((128, 1)), _full((128, 1))],
        out_specs=_per_b((128, 1)),
        out_shape=jax.ShapeDtypeStruct((BATCH, 128, 1), f32),
    )(pm4, s4, q4, g4c, b4c)

    pooled2 = pooled.reshape(BATCH, 128)
    out = pl.pallas_call(
        _f5_kernel,
        in_specs=[pl.BlockSpec((BATCH, 128), lambda: (0, 0)),
                  pl.BlockSpec((40, 128), lambda: (0, 0)),
                  pl.BlockSpec((8, 40), lambda: (0, 0))],
        out_specs=pl.BlockSpec((BATCH, 40), lambda: (0, 0)),
        out_shape=jax.ShapeDtypeStruct((BATCH, 40), f32),
    )(pooled2, Wl, bl8)
    return out
